# baseline (speedup 1.0000x reference)
import functools

import jax
import jax.numpy as jnp
from jax import lax
from jax.experimental import pallas as pl
from jax.experimental.pallas import tpu as pltpu
from jax.experimental.pallas import tpu_sc as plsc

_BATCH = 4096
_SEQ = 50
_DIM = 128
_N = _BATCH * _SEQ
_NW = 32
_NSUB = 16
_ROWS_PER_W = _N // _NW
_CHUNK = 160
_NCHUNK = _ROWS_PER_W // _CHUNK
_NBUF = 4
_NSP = 2


def _sc_gather(idx_flat, weight):
    mesh = plsc.VectorSubcoreMesh(core_axis_name="c", subcore_axis_name="s")

    @functools.partial(
        pl.kernel,
        mesh=mesh,
        out_type=jax.ShapeDtypeStruct((_N, _DIM), jnp.float32),
        scratch_types=[
            pltpu.VMEM((_ROWS_PER_W,), jnp.int32),
            pltpu.VMEM((_NBUF, _CHUNK, _DIM), jnp.float32),
            pltpu.VMEM_SHARED((_NSUB, _NSP, _CHUNK, _DIM), jnp.float32),
            pltpu.SemaphoreType.DMA((_NBUF,)),
            pltpu.SemaphoreType.DMA((_NBUF,)),
            pltpu.SemaphoreType.DMA((_NSP,)),
        ],
    )
    def k(idx_hbm, table_hbm, out_hbm, idx_v, rows_v, sp, gsem, dsem, ysem):
        cid = lax.axis_index("c")
        sid = lax.axis_index("s")
        wid = sid * 2 + cid
        base = wid * _ROWS_PER_W
        pltpu.sync_copy(idx_hbm.at[pl.ds(base, _ROWS_PER_W)], idx_v)

        def out_at(c):
            return out_hbm.at[pl.ds(base + c * _CHUNK, _CHUNK)]

        def start_gather(c, b):
            pltpu.make_async_copy(
                table_hbm.at[idx_v.at[pl.ds(c * _CHUNK, _CHUNK)]],
                rows_v.at[b], gsem.at[b]).start()

        def wait_gather(c, b):
            pltpu.make_async_copy(
                table_hbm.at[idx_v.at[pl.ds(c * _CHUNK, _CHUNK)]],
                rows_v.at[b], gsem.at[b]).wait()

        def start_direct(c):
            b = c % _NBUF
            pltpu.make_async_copy(rows_v.at[b], out_at(c), dsem.at[b]).start()

        def wait_direct(c):
            b = c % _NBUF
            pltpu.make_async_copy(rows_v.at[b], out_at(c), dsem.at[b]).wait()

        def stage(c):
            b = c % _NBUF
            m = (c // 2) % _NSP
            pltpu.sync_copy(rows_v.at[b], sp.at[sid, m])
            pltpu.make_async_copy(sp.at[sid, m], out_at(c), ysem.at[m]).start()

        def wait_staged(c):
            m = (c // 2) % _NSP
            pltpu.make_async_copy(sp.at[sid, m], out_at(c), ysem.at[m]).wait()

        start_gather(0, 0)
        start_gather(1, 1)
        wait_gather(0, 0)
        start_direct(0)
        start_gather(2, 2)
        wait_gather(1, 1)
        stage(1)
        start_gather(3, 3)
        wait_gather(2, 2)
        start_direct(2)

        def group(g, first):
            c0 = 4 * g
            wait_direct(c0 - 4)
            start_gather(c0, 0)
            wait_gather(c0 - 1, 3)
            if not first:
                wait_staged(c0 - 5)
            stage(c0 - 1)

            start_gather(c0 + 1, 1)
            wait_gather(c0, 0)
            start_direct(c0)

            wait_direct(c0 - 2)
            start_gather(c0 + 2, 2)
            wait_gather(c0 + 1, 1)
            wait_staged(c0 - 3)
            stage(c0 + 1)

            start_gather(c0 + 3, 3)
            wait_gather(c0 + 2, 2)
            start_direct(c0 + 2)

        group(1, first=True)

        def body(g, carry):
            group(g, first=False)
            return carry

        lax.fori_loop(2, _NCHUNK // _NBUF, body, 0)

        last = _NCHUNK - 1
        wait_gather(last, 3)
        wait_staged(last - 4)
        stage(last)
        wait_direct(last - 3)
        wait_direct(last - 1)
        wait_staged(last - 2)
        wait_staged(last)

    return k(idx_flat, weight)


def kernel(input_ids, weight):
    idx_flat = jnp.transpose(input_ids).reshape(_N).astype(jnp.int32)
    out_flat = _sc_gather(idx_flat, weight)
    return out_flat.reshape(_SEQ, _BATCH, _DIM)


# speedup vs baseline: 1.0617x; 1.0617x over previous
import functools

import jax
import jax.numpy as jnp
from jax import lax
from jax.experimental import pallas as pl
from jax.experimental.pallas import tpu as pltpu
from jax.experimental.pallas import tpu_sc as plsc

_BATCH = 4096
_SEQ = 50
_DIM = 128
_N = _BATCH * _SEQ
_NW = 32
_NSUB = 16
_ROWS_PER_W = _N // _NW
_CHUNK = 160
_NCHUNK = _ROWS_PER_W // _CHUNK
_NBUF = 4
_NSP = 2


def _sc_gather(idx_flat, weight):
    mesh = plsc.VectorSubcoreMesh(core_axis_name="c", subcore_axis_name="s")

    @functools.partial(
        pl.kernel,
        mesh=mesh,
        out_type=jax.ShapeDtypeStruct((_N, _DIM), jnp.float32),
        scratch_types=[
            pltpu.VMEM((_ROWS_PER_W,), jnp.int32),
            pltpu.VMEM((_NBUF, _CHUNK, _DIM), jnp.float32),
            pltpu.VMEM_SHARED((_NSUB, _NSP, _CHUNK, _DIM), jnp.float32),
            pltpu.SemaphoreType.DMA((_NBUF,)),
            pltpu.SemaphoreType.DMA((_NSP,)),
            pltpu.SemaphoreType.DMA((_NSP,)),
        ],
    )
    def k(idx_hbm, table_hbm, out_hbm, idx_v, rows_v, sp, gsem, xsem, ysem):
        cid = lax.axis_index("c")
        sid = lax.axis_index("s")
        wid = sid * 2 + cid
        base = wid * _ROWS_PER_W
        pltpu.sync_copy(idx_hbm.at[pl.ds(base, _ROWS_PER_W)], idx_v)

        def start_gather(c, b):
            pltpu.make_async_copy(
                table_hbm.at[idx_v.at[pl.ds(c * _CHUNK, _CHUNK)]],
                rows_v.at[b], gsem.at[b]).start()

        def wait_gather(c, b):
            pltpu.make_async_copy(
                table_hbm.at[idx_v.at[pl.ds(c * _CHUNK, _CHUNK)]],
                rows_v.at[b], gsem.at[b]).wait()

        def stage_x(c):
            pltpu.make_async_copy(
                rows_v.at[c % _NBUF], sp.at[sid, c % _NSP],
                xsem.at[c % _NSP]).start()

        def flush(c):
            m = c % _NSP
            pltpu.make_async_copy(
                rows_v.at[c % _NBUF], sp.at[sid, m], xsem.at[m]).wait()
            pltpu.make_async_copy(
                sp.at[sid, m],
                out_hbm.at[pl.ds(base + c * _CHUNK, _CHUNK)],
                ysem.at[m]).start()

        def wait_out(c):
            m = c % _NSP
            pltpu.make_async_copy(
                sp.at[sid, m],
                out_hbm.at[pl.ds(base + c * _CHUNK, _CHUNK)],
                ysem.at[m]).wait()

        start_gather(0, 0)
        start_gather(1, 1)
        wait_gather(0, 0)
        stage_x(0)
        start_gather(2, 2)
        wait_gather(1, 1)
        stage_x(1)
        flush(0)
        start_gather(3, 3)
        wait_gather(2, 2)
        wait_out(0)
        stage_x(2)
        flush(1)

        def group(g, carry):
            for b in range(_NBUF):
                c = g * _NBUF + b
                start_gather(c, b)
                wait_gather(c - 1, (b - 1) % _NBUF)
                wait_out(c - 3)
                stage_x(c - 1)
                flush(c - 2)
            return carry

        lax.fori_loop(1, _NCHUNK // _NBUF, group, 0)

        last = _NCHUNK - 1
        wait_gather(last, last % _NBUF)
        wait_out(last - 2)
        stage_x(last)
        flush(last - 1)
        flush(last)
        wait_out(last - 1)
        wait_out(last)

    return k(idx_flat, weight)


def kernel(input_ids, weight):
    idx_flat = jnp.transpose(input_ids).reshape(_N).astype(jnp.int32)
    out_flat = _sc_gather(idx_flat, weight)
    return out_flat.reshape(_SEQ, _BATCH, _DIM)
